# baseline (speedup 1.0000x reference)
import functools

import jax
import jax.numpy as jnp
from jax import lax
from jax.experimental import pallas as pl
from jax.experimental.pallas import tpu as pltpu
from jax.experimental.pallas import tpu_sc as plsc

B = 16384
K = 16
NC = 2
NS = 16
NW = NC * NS
BPW = B // NW
CHUNK = 128
NCHUNK = BPW // CHUNK
GROUPS = BPW // K


def _bpr_body(user_h, pos_h, neg_h, eu_h, ei_h, out_h,
              idx_u, idx_p, idx_n, u_rows, p_rows, n_rows, out_v, sem):
    wid = lax.axis_index("s") * NC + lax.axis_index("c")
    irow0 = wid * NCHUNK

    pltpu.sync_copy(user_h.at[pl.ds(irow0, NCHUNK)], idx_u)
    pltpu.sync_copy(pos_h.at[pl.ds(irow0, NCHUNK)], idx_p)
    pltpu.sync_copy(neg_h.at[pl.ds(irow0, NCHUNK)], idx_n)

    copies = []
    for j in range(NCHUNK):
        sl = pl.ds(j * CHUNK, CHUNK)
        copies.append(pltpu.async_copy(eu_h.at[idx_u.at[j]], u_rows.at[sl], sem))
        copies.append(pltpu.async_copy(ei_h.at[idx_p.at[j]], p_rows.at[sl], sem))
        copies.append(pltpu.async_copy(ei_h.at[idx_n.at[j]], n_rows.at[sl], sem))
    for c in copies:
        c.wait()

    lane = lax.iota(jnp.int32, 16)

    def group(g, carry):
        rows = g * K + lane
        acc = jnp.zeros((K,), jnp.float32)
        for k in range(K):
            col = jnp.full((K,), k, jnp.int32)
            uk = plsc.load_gather(u_rows, [rows, col])
            pk = plsc.load_gather(p_rows, [rows, col])
            nk = plsc.load_gather(n_rows, [rows, col])
            acc = acc + uk * (pk - nk)
        out_v[pl.ds(g * K, K)] = acc
        return carry

    lax.fori_loop(0, GROUPS, group, 0)

    pltpu.sync_copy(out_v, out_h.at[pl.ds(wid * BPW, BPW)])


@jax.jit
def kernel(user, pos_item, neg_item, embedding_user, embedding_item):
    mesh = plsc.VectorSubcoreMesh(core_axis_name="c", subcore_axis_name="s")
    f = pl.kernel(
        _bpr_body,
        out_type=jax.ShapeDtypeStruct((B,), jnp.float32),
        mesh=mesh,
        scratch_types=[
            pltpu.VMEM((NCHUNK, CHUNK), jnp.int32),
            pltpu.VMEM((NCHUNK, CHUNK), jnp.int32),
            pltpu.VMEM((NCHUNK, CHUNK), jnp.int32),
            pltpu.VMEM((BPW, K), jnp.float32),
            pltpu.VMEM((BPW, K), jnp.float32),
            pltpu.VMEM((BPW, K), jnp.float32),
            pltpu.VMEM((BPW,), jnp.float32),
            pltpu.SemaphoreType.DMA,
        ],
        compiler_params=pltpu.CompilerParams(
            needs_layout_passes=False, use_tc_tiling_on_sc=False),
    )
    u2 = user.astype(jnp.int32).reshape(NW * NCHUNK, CHUNK)
    p2 = pos_item.astype(jnp.int32).reshape(NW * NCHUNK, CHUNK)
    n2 = neg_item.astype(jnp.int32).reshape(NW * NCHUNK, CHUNK)
    return f(u2, p2, n2, embedding_user, embedding_item)


# speedup vs baseline: 1.0002x; 1.0002x over previous
import jax
import jax.numpy as jnp
from jax import lax
from jax.experimental import pallas as pl
from jax.experimental.pallas import tpu as pltpu
from jax.experimental.pallas import tpu_sc as plsc

B = 16384
K = 16
NC = 2
NS = 16
NW = NC * NS
BPW = B // NW
CHUNK = 128
NCHUNK = BPW // CHUNK
GROUPS = BPW // K
NROW = 1000000


def _bpr_body(user_h, pos_h, neg_h, eu_h, ei_h, out_h,
              idx_u, idx_p, idx_n, u_buf, p_buf, n_buf, out_v, sem):
    wid = lax.axis_index("s") * NC + lax.axis_index("c")
    irow0 = wid * NCHUNK

    pltpu.sync_copy(user_h.at[pl.ds(irow0, NCHUNK)], idx_u)
    pltpu.sync_copy(pos_h.at[pl.ds(irow0, NCHUNK)], idx_p)
    pltpu.sync_copy(neg_h.at[pl.ds(irow0, NCHUNK)], idx_n)

    copies = []
    for j in range(NCHUNK):
        dst = pl.ds(j * CHUNK, CHUNK)
        copies.append(pltpu.async_copy(
            eu_h.at[plsc.Indices(idx_u.at[j])], u_buf.at[dst], sem))
        copies.append(pltpu.async_copy(
            ei_h.at[plsc.Indices(idx_p.at[j])], p_buf.at[dst], sem))
        copies.append(pltpu.async_copy(
            ei_h.at[plsc.Indices(idx_n.at[j])], n_buf.at[dst], sem))
    for c in copies:
        c.wait()

    lane = lax.iota(jnp.int32, 16)

    def group(g, carry):
        rows = g * K + lane
        acc = jnp.zeros((K,), jnp.float32)
        for k in range(K):
            col = jnp.full((16,), k, jnp.int32)
            u_k = plsc.load_gather(u_buf, [rows, col])
            p_k = plsc.load_gather(p_buf, [rows, col])
            n_k = plsc.load_gather(n_buf, [rows, col])
            acc = acc + u_k * (p_k - n_k)
        out_v[pl.ds(g * K, K)] = acc
        return carry

    lax.fori_loop(0, GROUPS, group, 0)

    pltpu.sync_copy(out_v, out_h.at[pl.ds(wid * BPW, BPW)])


@jax.jit
def kernel(user, pos_item, neg_item, embedding_user, embedding_item):
    mesh = plsc.VectorSubcoreMesh(core_axis_name="c", subcore_axis_name="s")
    f = pl.kernel(
        _bpr_body,
        out_type=jax.ShapeDtypeStruct((B,), jnp.float32),
        mesh=mesh,
        scratch_types=[
            pltpu.VMEM((NCHUNK, CHUNK), jnp.int32),
            pltpu.VMEM((NCHUNK, CHUNK), jnp.int32),
            pltpu.VMEM((NCHUNK, CHUNK), jnp.int32),
            pltpu.VMEM((BPW, K), jnp.float32),
            pltpu.VMEM((BPW, K), jnp.float32),
            pltpu.VMEM((BPW, K), jnp.float32),
            pltpu.VMEM((BPW,), jnp.float32),
            pltpu.SemaphoreType.DMA,
        ],
        compiler_params=pltpu.CompilerParams(
            needs_layout_passes=False,
            use_tc_tiling_on_sc=False,
        ),
    )
    u2 = user.astype(jnp.int32).reshape(NW * NCHUNK, CHUNK)
    p2 = pos_item.astype(jnp.int32).reshape(NW * NCHUNK, CHUNK)
    n2 = neg_item.astype(jnp.int32).reshape(NW * NCHUNK, CHUNK)
    return f(u2, p2, n2, embedding_user, embedding_item)
